# Initial kernel scaffold; baseline (speedup 1.0000x reference)
import jax
import jax.numpy as jnp
from jax.experimental import pallas as pl


def kernel(boxes, scores):
    raise NotImplementedError("write your pallas kernel here")


# speedup vs baseline: 30.7967x; 30.7967x over previous
import jax
import jax.numpy as jnp
from jax.experimental import pallas as pl
from jax.experimental.pallas import tpu as pltpu

_NMS_T = 0.3
_B = 512


def _nms_body(x1_ref, y1_ref, x2_ref, y2_ref, s_ref, out_ref, keep_ref, m_ref):
    nb = x1_ref.shape[0]

    col = jax.lax.broadcasted_iota(jnp.int32, (_B, _B), 1)
    row = jax.lax.broadcasted_iota(jnp.int32, (_B, _B), 0)
    ut = (col > row).astype(jnp.float32)

    def iou_gt(ax1, ay1, ax2, ay2, aarea, bx1, by1, bx2, by2, barea):
        ix1 = jnp.maximum(ax1[:, None], bx1[None, :])
        iy1 = jnp.maximum(ay1[:, None], by1[None, :])
        ix2 = jnp.minimum(ax2[:, None], bx2[None, :])
        iy2 = jnp.minimum(ay2[:, None], by2[None, :])
        w = jnp.maximum(ix2 - ix1 + 1.0, 0.0)
        h = jnp.maximum(iy2 - iy1 + 1.0, 0.0)
        inter = w * h
        iou = inter / (aarea[:, None] + barea[None, :] - inter)
        return (iou > _NMS_T).astype(jnp.float32)

    def outer(k, carry):
        bx1 = x1_ref[k]
        by1 = y1_ref[k]
        bx2 = x2_ref[k]
        by2 = y2_ref[k]
        barea = (bx2 - bx1 + 1.0) * (by2 - by1 + 1.0)

        def cross(j, sup):
            ax1 = x1_ref[j]
            ay1 = y1_ref[j]
            ax2 = x2_ref[j]
            ay2 = y2_ref[j]
            aarea = (ax2 - ax1 + 1.0) * (ay2 - ay1 + 1.0)
            m = iou_gt(ax1, ay1, ax2, ay2, aarea, bx1, by1, bx2, by2, barea)
            kj = keep_ref[j]
            return jnp.maximum(sup, jnp.max(m * kj[:, None], axis=0))

        sup = jax.lax.fori_loop(0, k, cross, jnp.zeros((_B,), jnp.float32))

        m_ref[...] = iou_gt(bx1, by1, bx2, by2, barea,
                            bx1, by1, bx2, by2, barea) * ut
        lane = jax.lax.iota(jnp.int32, _B)

        def inner(i, kb):
            ki = jnp.max(jnp.where(lane == i, kb, 0.0))
            return kb * (1.0 - ki * m_ref[i])

        kb = jax.lax.fori_loop(0, _B, inner, 1.0 - sup)
        keep_ref[k] = kb
        out_ref[k] = s_ref[k] * kb
        return carry

    jax.lax.fori_loop(0, nb, outer, 0)


def kernel(boxes, scores):
    n = scores.shape[0]
    order = jnp.argsort(-scores)
    b = boxes[order]
    s = scores[order]

    nb = -(-n // _B)
    npad = nb * _B - n
    far = 4.0e6
    x1 = jnp.pad(b[:, 0], (0, npad), constant_values=far).reshape(nb, _B)
    y1 = jnp.pad(b[:, 1], (0, npad), constant_values=far).reshape(nb, _B)
    x2 = jnp.pad(b[:, 2], (0, npad), constant_values=far + 1.0).reshape(nb, _B)
    y2 = jnp.pad(b[:, 3], (0, npad), constant_values=far + 1.0).reshape(nb, _B)
    sp = jnp.pad(s, (0, npad)).reshape(nb, _B)

    out = pl.pallas_call(
        _nms_body,
        out_shape=jax.ShapeDtypeStruct((nb, _B), jnp.float32),
        scratch_shapes=[
            pltpu.VMEM((nb, _B), jnp.float32),
            pltpu.VMEM((_B, _B), jnp.float32),
        ],
    )(x1, y1, x2, y2, sp)

    masked_sorted = out.reshape(nb * _B)[:n]
    return jnp.zeros((n,), scores.dtype).at[order].set(masked_sorted)


# speedup vs baseline: 31.5364x; 1.0240x over previous
import jax
import jax.numpy as jnp
from jax.experimental import pallas as pl
from jax.experimental.pallas import tpu as pltpu

_NMS_T = 0.3
_B = 512
_C = 128


def _nms_body(x1_ref, y1_ref, x2_ref, y2_ref, s_ref, out_ref, keep_ref, m_ref):
    nb = x1_ref.shape[0]

    col = jax.lax.broadcasted_iota(jnp.int32, (_B, _B), 1)
    row = jax.lax.broadcasted_iota(jnp.int32, (_B, _B), 0)
    ut = (col > row).astype(jnp.float32)

    def iou_gt(ax1, ay1, ax2, ay2, aarea, bx1, by1, bx2, by2, barea):
        ix1 = jnp.maximum(ax1[:, None], bx1[None, :])
        iy1 = jnp.maximum(ay1[:, None], by1[None, :])
        ix2 = jnp.minimum(ax2[:, None], bx2[None, :])
        iy2 = jnp.minimum(ay2[:, None], by2[None, :])
        w = jnp.maximum(ix2 - ix1 + 1.0, 0.0)
        h = jnp.maximum(iy2 - iy1 + 1.0, 0.0)
        inter = w * h
        iou = inter / (aarea[:, None] + barea[None, :] - inter)
        return (iou > _NMS_T).astype(jnp.float32)

    def outer(k, carry):
        bx1 = x1_ref[k]
        by1 = y1_ref[k]
        bx2 = x2_ref[k]
        by2 = y2_ref[k]
        barea = (bx2 - bx1 + 1.0) * (by2 - by1 + 1.0)

        def cross(j, cnt):
            ax1 = x1_ref[j]
            ay1 = y1_ref[j]
            ax2 = x2_ref[j]
            ay2 = y2_ref[j]
            aarea = (ax2 - ax1 + 1.0) * (ay2 - ay1 + 1.0)
            m = iou_gt(ax1, ay1, ax2, ay2, aarea, bx1, by1, bx2, by2, barea)
            kj = keep_ref[j]
            return cnt + jnp.dot(kj.reshape(1, _B), m,
                                 preferred_element_type=jnp.float32)

        cnt = jax.lax.fori_loop(0, k, cross, jnp.zeros((1, _B), jnp.float32))
        sup = (cnt > 0.0).astype(jnp.float32).reshape(_B)

        m_ref[...] = iou_gt(bx1, by1, bx2, by2, barea,
                            bx1, by1, bx2, by2, barea) * ut
        lane = jax.lax.iota(jnp.int32, _C)

        kb = 1.0 - sup
        kcs = []
        for c in range(_B // _C):
            kc = jax.lax.slice(kb, (c * _C,), ((c + 1) * _C,))

            def inner(i, kc):
                ki = jnp.max(jnp.where(lane == i, kc, 0.0))
                row = jax.lax.slice(m_ref[c * _C + i],
                                    (c * _C,), ((c + 1) * _C,))
                return kc * (1.0 - ki * row)

            kc = jax.lax.fori_loop(0, _C, inner, kc)
            kcs.append(kc)
            if c + 1 < _B // _C:
                rows = m_ref[pl.ds(c * _C, _C), :]
                pcnt = jnp.dot(kc.reshape(1, _C), rows,
                               preferred_element_type=jnp.float32)
                kb = kb * (1.0 - (pcnt > 0.0).astype(jnp.float32).reshape(_B))

        kb = jnp.concatenate(kcs)
        keep_ref[k] = kb
        out_ref[k] = s_ref[k] * kb
        return carry

    jax.lax.fori_loop(0, nb, outer, 0)


def kernel(boxes, scores):
    n = scores.shape[0]
    order = jnp.argsort(-scores)
    b = boxes[order]
    s = scores[order]

    nb = -(-n // _B)
    npad = nb * _B - n
    far = 4.0e6
    x1 = jnp.pad(b[:, 0], (0, npad), constant_values=far).reshape(nb, _B)
    y1 = jnp.pad(b[:, 1], (0, npad), constant_values=far).reshape(nb, _B)
    x2 = jnp.pad(b[:, 2], (0, npad), constant_values=far + 1.0).reshape(nb, _B)
    y2 = jnp.pad(b[:, 3], (0, npad), constant_values=far + 1.0).reshape(nb, _B)
    sp = jnp.pad(s, (0, npad)).reshape(nb, _B)

    out = pl.pallas_call(
        _nms_body,
        out_shape=jax.ShapeDtypeStruct((nb, _B), jnp.float32),
        scratch_shapes=[
            pltpu.VMEM((nb, _B), jnp.float32),
            pltpu.VMEM((_B, _B), jnp.float32),
        ],
    )(x1, y1, x2, y2, sp)

    masked_sorted = out.reshape(nb * _B)[:n]
    return jnp.zeros((n,), scores.dtype).at[order].set(masked_sorted)


# speedup vs baseline: 114.5354x; 3.6319x over previous
import jax
import jax.numpy as jnp
from jax.experimental import pallas as pl
from jax.experimental.pallas import tpu as pltpu

_NMS_T = 0.3
_B = 512
_C = 128


def _nms_body(x1_ref, y1_ref, x2_ref, y2_ref, s_ref, out_ref, keep_ref, m_ref):
    nb = x1_ref.shape[0]

    col = jax.lax.broadcasted_iota(jnp.int32, (_B, _B), 1)
    row = jax.lax.broadcasted_iota(jnp.int32, (_B, _B), 0)
    ut = (col > row).astype(jnp.float32)

    def iou_gt(ax1, ay1, ax2, ay2, aarea, bx1, by1, bx2, by2, barea):
        ix1 = jnp.maximum(ax1[:, None], bx1[None, :])
        iy1 = jnp.maximum(ay1[:, None], by1[None, :])
        ix2 = jnp.minimum(ax2[:, None], bx2[None, :])
        iy2 = jnp.minimum(ay2[:, None], by2[None, :])
        w = jnp.maximum(ix2 - ix1 + 1.0, 0.0)
        h = jnp.maximum(iy2 - iy1 + 1.0, 0.0)
        inter = w * h
        iou = inter / (aarea[:, None] + barea[None, :] - inter)
        return (iou > _NMS_T).astype(jnp.float32)

    def outer(k, carry):
        bx1 = x1_ref[k]
        by1 = y1_ref[k]
        bx2 = x2_ref[k]
        by2 = y2_ref[k]
        barea = (bx2 - bx1 + 1.0) * (by2 - by1 + 1.0)

        def cross(j, cnt):
            ax1 = x1_ref[j]
            ay1 = y1_ref[j]
            ax2 = x2_ref[j]
            ay2 = y2_ref[j]
            aarea = (ax2 - ax1 + 1.0) * (ay2 - ay1 + 1.0)
            m = iou_gt(ax1, ay1, ax2, ay2, aarea, bx1, by1, bx2, by2, barea)
            kj = keep_ref[j]
            return cnt + jnp.dot(kj.reshape(1, _B), m,
                                 preferred_element_type=jnp.float32)

        cnt = jax.lax.fori_loop(0, k, cross, jnp.zeros((1, _B), jnp.float32))
        sup = (cnt > 0.0).astype(jnp.float32).reshape(_B)

        mut = iou_gt(bx1, by1, bx2, by2, barea,
                     bx1, by1, bx2, by2, barea) * ut
        m_ref[...] = mut

        kb0 = (1.0 - sup).reshape(1, _B)

        def fx_cond(carry):
            k_old, k_new = carry
            return jnp.any(k_old != k_new)

        def fx_body(carry):
            _, k = carry
            cnt = jnp.dot(k, m_ref[...], preferred_element_type=jnp.float32)
            return k, kb0 * (cnt == 0.0).astype(jnp.float32)

        k1 = kb0 * (jnp.dot(kb0, m_ref[...],
                            preferred_element_type=jnp.float32) == 0.0)
        _, kb2 = jax.lax.while_loop(fx_cond, fx_body, (kb0, k1))
        kb = kb2.reshape(_B)
        keep_ref[k] = kb
        out_ref[k] = s_ref[k] * kb
        return carry

    jax.lax.fori_loop(0, nb, outer, 0)


def kernel(boxes, scores):
    n = scores.shape[0]
    order = jnp.argsort(-scores)
    b = boxes[order]
    s = scores[order]

    nb = -(-n // _B)
    npad = nb * _B - n
    far = 4.0e6
    x1 = jnp.pad(b[:, 0], (0, npad), constant_values=far).reshape(nb, _B)
    y1 = jnp.pad(b[:, 1], (0, npad), constant_values=far).reshape(nb, _B)
    x2 = jnp.pad(b[:, 2], (0, npad), constant_values=far + 1.0).reshape(nb, _B)
    y2 = jnp.pad(b[:, 3], (0, npad), constant_values=far + 1.0).reshape(nb, _B)
    sp = jnp.pad(s, (0, npad)).reshape(nb, _B)

    out = pl.pallas_call(
        _nms_body,
        out_shape=jax.ShapeDtypeStruct((nb, _B), jnp.float32),
        scratch_shapes=[
            pltpu.VMEM((nb, _B), jnp.float32),
            pltpu.VMEM((_B, _B), jnp.float32),
        ],
    )(x1, y1, x2, y2, sp)

    masked_sorted = out.reshape(nb * _B)[:n]
    return jnp.zeros((n,), scores.dtype).at[order].set(masked_sorted)


# speedup vs baseline: 119.9548x; 1.0473x over previous
import functools

import jax
import jax.numpy as jnp
from jax import lax
from jax.experimental import pallas as pl
from jax.experimental.pallas import tpu as pltpu
from jax.experimental.pallas import tpu_sc as plsc

_NMS_T = 0.3
_B = 512
_W = 128

_NC = 2
_NS = 16
_NW = _NC * _NS


def _sc_gather(np_rows):
    per_w = np_rows // _NW
    mesh = plsc.VectorSubcoreMesh(core_axis_name="c", subcore_axis_name="s")

    @functools.partial(
        pl.kernel, mesh=mesh,
        out_type=jax.ShapeDtypeStruct((np_rows, _W), jnp.float32),
        scratch_types=[
            pltpu.VMEM((per_w,), jnp.int32),
            pltpu.VMEM((per_w, _W), jnp.float32),
            pltpu.SemaphoreType.DMA,
        ],
    )
    def k(table_hbm, idx_hbm, out_hbm, idx_v, rows_v, sem):
        wid = lax.axis_index("s") * _NC + lax.axis_index("c")
        base = wid * per_w
        pltpu.sync_copy(idx_hbm.at[pl.ds(base, per_w)], idx_v)
        pltpu.async_copy(table_hbm.at[idx_v], rows_v, sem).wait()
        pltpu.sync_copy(rows_v, out_hbm.at[pl.ds(base, per_w)])

    return k


def _sc_scatter(np_rows):
    per_w = np_rows // _NW
    mesh = plsc.VectorSubcoreMesh(core_axis_name="c", subcore_axis_name="s")

    @functools.partial(
        pl.kernel, mesh=mesh,
        out_type=jax.ShapeDtypeStruct((np_rows, _W), jnp.float32),
        scratch_types=[
            pltpu.VMEM((per_w,), jnp.int32),
            pltpu.VMEM((per_w, _W), jnp.float32),
            pltpu.SemaphoreType.DMA,
        ],
    )
    def k(vals_hbm, idx_hbm, out_hbm, idx_v, rows_v, sem):
        wid = lax.axis_index("s") * _NC + lax.axis_index("c")
        base = wid * per_w
        pltpu.sync_copy(idx_hbm.at[pl.ds(base, per_w)], idx_v)
        pltpu.sync_copy(vals_hbm.at[pl.ds(base, per_w)], rows_v)
        pltpu.async_copy(rows_v, out_hbm.at[idx_v], sem).wait()

    return k


def _nms_body(x1_ref, y1_ref, x2_ref, y2_ref, s_ref, out_ref, keep_ref, m_ref):
    nb = x1_ref.shape[0]

    col = jax.lax.broadcasted_iota(jnp.int32, (_B, _B), 1)
    row = jax.lax.broadcasted_iota(jnp.int32, (_B, _B), 0)
    ut = (col > row).astype(jnp.float32)

    def iou_gt(ax1, ay1, ax2, ay2, aarea, bx1, by1, bx2, by2, barea):
        ix1 = jnp.maximum(ax1[:, None], bx1[None, :])
        iy1 = jnp.maximum(ay1[:, None], by1[None, :])
        ix2 = jnp.minimum(ax2[:, None], bx2[None, :])
        iy2 = jnp.minimum(ay2[:, None], by2[None, :])
        w = jnp.maximum(ix2 - ix1 + 1.0, 0.0)
        h = jnp.maximum(iy2 - iy1 + 1.0, 0.0)
        inter = w * h
        iou = inter / (aarea[:, None] + barea[None, :] - inter)
        return (iou > _NMS_T).astype(jnp.float32)

    def outer(k, carry):
        bx1 = x1_ref[k]
        by1 = y1_ref[k]
        bx2 = x2_ref[k]
        by2 = y2_ref[k]
        barea = (bx2 - bx1 + 1.0) * (by2 - by1 + 1.0)

        def cross(j, cnt):
            ax1 = x1_ref[j]
            ay1 = y1_ref[j]
            ax2 = x2_ref[j]
            ay2 = y2_ref[j]
            aarea = (ax2 - ax1 + 1.0) * (ay2 - ay1 + 1.0)
            m = iou_gt(ax1, ay1, ax2, ay2, aarea, bx1, by1, bx2, by2, barea)
            kj = keep_ref[j]
            return cnt + jnp.dot(kj.reshape(1, _B), m,
                                 preferred_element_type=jnp.float32)

        cnt = jax.lax.fori_loop(0, k, cross, jnp.zeros((1, _B), jnp.float32))
        sup = (cnt > 0.0).astype(jnp.float32).reshape(_B)

        mut = iou_gt(bx1, by1, bx2, by2, barea,
                     bx1, by1, bx2, by2, barea) * ut
        m_ref[...] = mut

        kb0 = (1.0 - sup).reshape(1, _B)

        def fx_cond(carry):
            k_old, k_new = carry
            return jnp.any(k_old != k_new)

        def fx_body(carry):
            _, k = carry
            cnt = jnp.dot(k, m_ref[...], preferred_element_type=jnp.float32)
            return k, kb0 * (cnt == 0.0).astype(jnp.float32)

        k1 = kb0 * (jnp.dot(kb0, m_ref[...],
                            preferred_element_type=jnp.float32) == 0.0)
        _, kb2 = jax.lax.while_loop(fx_cond, fx_body, (kb0, k1))
        kb = kb2.reshape(_B)
        keep_ref[k] = kb
        out_ref[k] = s_ref[k] * kb
        return carry

    jax.lax.fori_loop(0, nb, outer, 0)


def kernel(boxes, scores):
    n = scores.shape[0]
    order = jnp.argsort(-scores)

    nb = -(-n // _B)
    np_rows = nb * _B
    npad = np_rows - n
    far = 4.0e6

    pad_row = jnp.array([far, far, far + 1.0, far + 1.0] + [0.0] * (_W - 4),
                        jnp.float32)
    table = jnp.concatenate(
        [boxes.astype(jnp.float32), scores[:, None].astype(jnp.float32),
         jnp.zeros((n, _W - 5), jnp.float32)], axis=1)
    table = jnp.concatenate(
        [table, jnp.broadcast_to(pad_row, (npad, _W))], axis=0)
    idx = jnp.concatenate(
        [order.astype(jnp.int32),
         jnp.arange(n, np_rows, dtype=jnp.int32)], axis=0)

    g = _sc_gather(np_rows)(table, idx)

    x1 = g[:, 0].reshape(nb, _B)
    y1 = g[:, 1].reshape(nb, _B)
    x2 = g[:, 2].reshape(nb, _B)
    y2 = g[:, 3].reshape(nb, _B)
    sp = g[:, 4].reshape(nb, _B)

    out = pl.pallas_call(
        _nms_body,
        out_shape=jax.ShapeDtypeStruct((nb, _B), jnp.float32),
        scratch_shapes=[
            pltpu.VMEM((nb, _B), jnp.float32),
            pltpu.VMEM((_B, _B), jnp.float32),
        ],
    )(x1, y1, x2, y2, sp)

    vals = jnp.concatenate(
        [out.reshape(np_rows)[:, None],
         jnp.zeros((np_rows, _W - 1), jnp.float32)], axis=1)
    scat = _sc_scatter(np_rows)(vals, idx)
    return scat[:n, 0].astype(scores.dtype)


# speedup vs baseline: 146.0148x; 1.2172x over previous
import functools

import jax
import jax.numpy as jnp
from jax import lax
from jax.experimental import pallas as pl
from jax.experimental.pallas import tpu as pltpu
from jax.experimental.pallas import tpu_sc as plsc

_NMS_T = 0.3
_B = 1024
_W = 128

_NC = 2
_NS = 16
_NW = _NC * _NS


def _sc_gather(np_rows):
    per_w = np_rows // _NW
    mesh = plsc.VectorSubcoreMesh(core_axis_name="c", subcore_axis_name="s")

    @functools.partial(
        pl.kernel, mesh=mesh,
        out_type=jax.ShapeDtypeStruct((np_rows, _W), jnp.float32),
        scratch_types=[
            pltpu.VMEM((per_w,), jnp.int32),
            pltpu.VMEM((per_w, _W), jnp.float32),
            pltpu.SemaphoreType.DMA,
        ],
    )
    def k(table_hbm, idx_hbm, out_hbm, idx_v, rows_v, sem):
        wid = lax.axis_index("s") * _NC + lax.axis_index("c")
        base = wid * per_w
        pltpu.sync_copy(idx_hbm.at[pl.ds(base, per_w)], idx_v)
        pltpu.async_copy(table_hbm.at[idx_v], rows_v, sem).wait()
        pltpu.sync_copy(rows_v, out_hbm.at[pl.ds(base, per_w)])

    return k


def _sc_scatter(np_rows):
    per_w = np_rows // _NW
    mesh = plsc.VectorSubcoreMesh(core_axis_name="c", subcore_axis_name="s")

    @functools.partial(
        pl.kernel, mesh=mesh,
        out_type=jax.ShapeDtypeStruct((np_rows, _W), jnp.float32),
        scratch_types=[
            pltpu.VMEM((per_w,), jnp.int32),
            pltpu.VMEM((per_w, _W), jnp.float32),
            pltpu.SemaphoreType.DMA,
        ],
    )
    def k(vals_hbm, idx_hbm, out_hbm, idx_v, rows_v, sem):
        wid = lax.axis_index("s") * _NC + lax.axis_index("c")
        base = wid * per_w
        pltpu.sync_copy(idx_hbm.at[pl.ds(base, per_w)], idx_v)
        pltpu.sync_copy(vals_hbm.at[pl.ds(base, per_w)], rows_v)
        pltpu.async_copy(rows_v, out_hbm.at[idx_v], sem).wait()

    return k


def _nms_body(x1_ref, y1_ref, x2_ref, y2_ref, s_ref, out_ref, keep_ref, m_ref):
    nb = x1_ref.shape[0]

    col = jax.lax.broadcasted_iota(jnp.int32, (_B, _B), 1)
    row = jax.lax.broadcasted_iota(jnp.int32, (_B, _B), 0)
    ut = (col > row).astype(jnp.float32)

    def iou_gt(ax1, ay1, ax2, ay2, aarea, bx1, by1, bx2, by2, barea):
        ix1 = jnp.maximum(ax1[:, None], bx1[None, :])
        iy1 = jnp.maximum(ay1[:, None], by1[None, :])
        ix2 = jnp.minimum(ax2[:, None], bx2[None, :])
        iy2 = jnp.minimum(ay2[:, None], by2[None, :])
        w = jnp.maximum(ix2 - ix1 + 1.0, 0.0)
        h = jnp.maximum(iy2 - iy1 + 1.0, 0.0)
        inter = w * h
        iou = inter / (aarea[:, None] + barea[None, :] - inter)
        return (iou > _NMS_T).astype(jnp.float32)

    def outer(k, carry):
        bx1 = x1_ref[k]
        by1 = y1_ref[k]
        bx2 = x2_ref[k]
        by2 = y2_ref[k]
        barea = (bx2 - bx1 + 1.0) * (by2 - by1 + 1.0)

        def cross(j, cnt):
            ax1 = x1_ref[j]
            ay1 = y1_ref[j]
            ax2 = x2_ref[j]
            ay2 = y2_ref[j]
            aarea = (ax2 - ax1 + 1.0) * (ay2 - ay1 + 1.0)
            m = iou_gt(ax1, ay1, ax2, ay2, aarea, bx1, by1, bx2, by2, barea)
            kj = keep_ref[j]
            return cnt + jnp.dot(kj.reshape(1, _B), m,
                                 preferred_element_type=jnp.float32)

        cnt = jax.lax.fori_loop(0, k, cross, jnp.zeros((1, _B), jnp.float32))
        sup = (cnt > 0.0).astype(jnp.float32).reshape(_B)

        mut = iou_gt(bx1, by1, bx2, by2, barea,
                     bx1, by1, bx2, by2, barea) * ut
        m_ref[...] = mut

        kb0 = (1.0 - sup).reshape(1, _B)

        def fx_cond(carry):
            k_old, k_new = carry
            return jnp.any(k_old != k_new)

        def fx_body(carry):
            _, k = carry
            cnt = jnp.dot(k, m_ref[...], preferred_element_type=jnp.float32)
            return k, kb0 * (cnt == 0.0).astype(jnp.float32)

        k1 = kb0 * (jnp.dot(kb0, m_ref[...],
                            preferred_element_type=jnp.float32) == 0.0)
        _, kb2 = jax.lax.while_loop(fx_cond, fx_body, (kb0, k1))
        kb = kb2.reshape(_B)
        keep_ref[k] = kb
        out_ref[k] = s_ref[k] * kb
        return carry

    jax.lax.fori_loop(0, nb, outer, 0)


def kernel(boxes, scores):
    n = scores.shape[0]
    order = jnp.argsort(-scores)

    nb = -(-n // _B)
    np_rows = nb * _B
    npad = np_rows - n
    far = 4.0e6

    pad_row = jnp.array([far, far, far + 1.0, far + 1.0] + [0.0] * (_W - 4),
                        jnp.float32)
    table = jnp.concatenate(
        [boxes.astype(jnp.float32), scores[:, None].astype(jnp.float32),
         jnp.zeros((n, _W - 5), jnp.float32)], axis=1)
    table = jnp.concatenate(
        [table, jnp.broadcast_to(pad_row, (npad, _W))], axis=0)
    idx = jnp.concatenate(
        [order.astype(jnp.int32),
         jnp.arange(n, np_rows, dtype=jnp.int32)], axis=0)

    g = _sc_gather(np_rows)(table, idx)

    x1 = g[:, 0].reshape(nb, _B)
    y1 = g[:, 1].reshape(nb, _B)
    x2 = g[:, 2].reshape(nb, _B)
    y2 = g[:, 3].reshape(nb, _B)
    sp = g[:, 4].reshape(nb, _B)

    out = pl.pallas_call(
        _nms_body,
        out_shape=jax.ShapeDtypeStruct((nb, _B), jnp.float32),
        scratch_shapes=[
            pltpu.VMEM((nb, _B), jnp.float32),
            pltpu.VMEM((_B, _B), jnp.float32),
        ],
    )(x1, y1, x2, y2, sp)

    vals = jnp.concatenate(
        [out.reshape(np_rows)[:, None],
         jnp.zeros((np_rows, _W - 1), jnp.float32)], axis=1)
    scat = _sc_scatter(np_rows)(vals, idx)
    return scat[:n, 0].astype(scores.dtype)


# speedup vs baseline: 152.1669x; 1.0421x over previous
import functools

import jax
import jax.numpy as jnp
from jax import lax
from jax.experimental import pallas as pl
from jax.experimental.pallas import tpu as pltpu
from jax.experimental.pallas import tpu_sc as plsc

_NMS_T = 0.3
_B = 2048
_W = 128

_NC = 2
_NS = 16
_NW = _NC * _NS


def _sc_gather(np_rows):
    per_w = np_rows // _NW
    mesh = plsc.VectorSubcoreMesh(core_axis_name="c", subcore_axis_name="s")

    @functools.partial(
        pl.kernel, mesh=mesh,
        out_type=jax.ShapeDtypeStruct((np_rows, _W), jnp.float32),
        scratch_types=[
            pltpu.VMEM((per_w,), jnp.int32),
            pltpu.VMEM((per_w, _W), jnp.float32),
            pltpu.SemaphoreType.DMA,
        ],
    )
    def k(table_hbm, idx_hbm, out_hbm, idx_v, rows_v, sem):
        wid = lax.axis_index("s") * _NC + lax.axis_index("c")
        base = wid * per_w
        pltpu.sync_copy(idx_hbm.at[pl.ds(base, per_w)], idx_v)
        pltpu.async_copy(table_hbm.at[idx_v], rows_v, sem).wait()
        pltpu.sync_copy(rows_v, out_hbm.at[pl.ds(base, per_w)])

    return k


def _sc_scatter(np_rows):
    per_w = np_rows // _NW
    mesh = plsc.VectorSubcoreMesh(core_axis_name="c", subcore_axis_name="s")

    @functools.partial(
        pl.kernel, mesh=mesh,
        out_type=jax.ShapeDtypeStruct((np_rows, _W), jnp.float32),
        scratch_types=[
            pltpu.VMEM((per_w,), jnp.int32),
            pltpu.VMEM((per_w, _W), jnp.float32),
            pltpu.SemaphoreType.DMA,
        ],
    )
    def k(vals_hbm, idx_hbm, out_hbm, idx_v, rows_v, sem):
        wid = lax.axis_index("s") * _NC + lax.axis_index("c")
        base = wid * per_w
        pltpu.sync_copy(idx_hbm.at[pl.ds(base, per_w)], idx_v)
        pltpu.sync_copy(vals_hbm.at[pl.ds(base, per_w)], rows_v)
        pltpu.async_copy(rows_v, out_hbm.at[idx_v], sem).wait()

    return k


def _nms_body(x1_ref, y1_ref, x2_ref, y2_ref, s_ref, out_ref, keep_ref, m_ref):
    nb = x1_ref.shape[0]

    col = jax.lax.broadcasted_iota(jnp.int32, (_B, _B), 1)
    row = jax.lax.broadcasted_iota(jnp.int32, (_B, _B), 0)
    ut = (col > row).astype(jnp.float32)

    def iou_gt(ax1, ay1, ax2, ay2, aarea, bx1, by1, bx2, by2, barea):
        ix1 = jnp.maximum(ax1[:, None], bx1[None, :])
        iy1 = jnp.maximum(ay1[:, None], by1[None, :])
        ix2 = jnp.minimum(ax2[:, None], bx2[None, :])
        iy2 = jnp.minimum(ay2[:, None], by2[None, :])
        w = jnp.maximum(ix2 - ix1 + 1.0, 0.0)
        h = jnp.maximum(iy2 - iy1 + 1.0, 0.0)
        inter = w * h
        iou = inter / (aarea[:, None] + barea[None, :] - inter)
        return (iou > _NMS_T).astype(jnp.float32)

    def outer(k, carry):
        bx1 = x1_ref[k]
        by1 = y1_ref[k]
        bx2 = x2_ref[k]
        by2 = y2_ref[k]
        barea = (bx2 - bx1 + 1.0) * (by2 - by1 + 1.0)

        def cross(j, cnt):
            ax1 = x1_ref[j]
            ay1 = y1_ref[j]
            ax2 = x2_ref[j]
            ay2 = y2_ref[j]
            aarea = (ax2 - ax1 + 1.0) * (ay2 - ay1 + 1.0)
            m = iou_gt(ax1, ay1, ax2, ay2, aarea, bx1, by1, bx2, by2, barea)
            kj = keep_ref[j]
            return cnt + jnp.dot(kj.reshape(1, _B), m,
                                 preferred_element_type=jnp.float32)

        cnt = jax.lax.fori_loop(0, k, cross, jnp.zeros((1, _B), jnp.float32))
        sup = (cnt > 0.0).astype(jnp.float32).reshape(_B)

        mut = iou_gt(bx1, by1, bx2, by2, barea,
                     bx1, by1, bx2, by2, barea) * ut
        m_ref[...] = mut

        kb0 = (1.0 - sup).reshape(1, _B)

        def fx_cond(carry):
            k_old, k_new = carry
            return jnp.any(k_old != k_new)

        def fx_body(carry):
            _, k = carry
            cnt = jnp.dot(k, m_ref[...], preferred_element_type=jnp.float32)
            return k, kb0 * (cnt == 0.0).astype(jnp.float32)

        k1 = kb0 * (jnp.dot(kb0, m_ref[...],
                            preferred_element_type=jnp.float32) == 0.0)
        _, kb2 = jax.lax.while_loop(fx_cond, fx_body, (kb0, k1))
        kb = kb2.reshape(_B)
        keep_ref[k] = kb
        out_ref[k] = s_ref[k] * kb
        return carry

    jax.lax.fori_loop(0, nb, outer, 0)


def kernel(boxes, scores):
    n = scores.shape[0]
    order = jnp.argsort(-scores)

    nb = -(-n // _B)
    np_rows = nb * _B
    npad = np_rows - n
    far = 4.0e6

    pad_row = jnp.array([far, far, far + 1.0, far + 1.0] + [0.0] * (_W - 4),
                        jnp.float32)
    table = jnp.concatenate(
        [boxes.astype(jnp.float32), scores[:, None].astype(jnp.float32),
         jnp.zeros((n, _W - 5), jnp.float32)], axis=1)
    table = jnp.concatenate(
        [table, jnp.broadcast_to(pad_row, (npad, _W))], axis=0)
    idx = jnp.concatenate(
        [order.astype(jnp.int32),
         jnp.arange(n, np_rows, dtype=jnp.int32)], axis=0)

    g = _sc_gather(np_rows)(table, idx)

    x1 = g[:, 0].reshape(nb, _B)
    y1 = g[:, 1].reshape(nb, _B)
    x2 = g[:, 2].reshape(nb, _B)
    y2 = g[:, 3].reshape(nb, _B)
    sp = g[:, 4].reshape(nb, _B)

    out = pl.pallas_call(
        _nms_body,
        out_shape=jax.ShapeDtypeStruct((nb, _B), jnp.float32),
        scratch_shapes=[
            pltpu.VMEM((nb, _B), jnp.float32),
            pltpu.VMEM((_B, _B), jnp.float32),
        ],
    )(x1, y1, x2, y2, sp)

    vals = jnp.concatenate(
        [out.reshape(np_rows)[:, None],
         jnp.zeros((np_rows, _W - 1), jnp.float32)], axis=1)
    scat = _sc_scatter(np_rows)(vals, idx)
    return scat[:n, 0].astype(scores.dtype)
